# baseline (speedup 1.0000x reference)
import functools

import jax
import jax.numpy as jnp
from jax import lax
from jax.experimental import pallas as pl
from jax.experimental.pallas import tpu as pltpu
from jax.experimental.pallas import tpu_sc as plsc

N_ROWS = 16384
N_CLASSES = 1000
N_STAGED_ROWS = 1024
NC = 2
NS = 16
NW = NC * NS
B_PER_W = N_ROWS // NW
CHUNK = 128
N_CHUNKS = B_PER_W // CHUNK
LANES = 16
SUB = CHUNK // LANES


def _build_kernel():
  mesh = plsc.VectorSubcoreMesh(core_axis_name="c", subcore_axis_name="s")

  @functools.partial(
      pl.kernel,
      mesh=mesh,
      out_type=jax.ShapeDtypeStruct((NW, LANES), jnp.float32),
      scratch_types=[
          pltpu.VMEM((B_PER_W,), jnp.int32),
          pltpu.VMEM((N_CHUNKS, CHUNK), jnp.int32),
          pltpu.VMEM((N_CHUNKS, CHUNK), jnp.float32),
          pltpu.VMEM((LANES,), jnp.float32),
          pltpu.SemaphoreType.DMA,
      ],
  )
  def loss_kernel(flat_hbm, tgt_hbm, out_hbm, tgt_v, idx_v, vals_v, acc_v, sem):
    wid = lax.axis_index("s") * NC + lax.axis_index("c")
    base = wid * B_PER_W
    pltpu.sync_copy(tgt_hbm.at[pl.ds(base, B_PER_W)], tgt_v)
    copies = []
    for j in range(N_CHUNKS):
      for k in range(SUB):
        t = tgt_v[pl.ds(j * CHUNK + k * LANES, LANES)]
        i = base + j * CHUNK + k * LANES + lax.iota(jnp.int32, LANES)
        q = lax.shift_right_logical(i * 67109, 26)
        rem = i - q * N_CLASSES
        rr = t + q
        idx_v[j, pl.ds(k * LANES, LANES)] = (
            lax.shift_left(lax.shift_right_logical(rem, 3), 17)
            + lax.shift_left(lax.shift_right_logical(rr, 7), 10)
            + lax.shift_left(lax.bitwise_and(rem, 7), 7)
            + lax.bitwise_and(rr, 127)
        )
      copies.append(
          pltpu.async_copy(flat_hbm.at[idx_v.at[j]], vals_v.at[j], sem))
    acc = jnp.zeros((LANES,), jnp.float32)
    for j in range(N_CHUNKS):
      copies[j].wait()
      for k in range(SUB):
        acc = acc + vals_v[j, pl.ds(k * LANES, LANES)]
    acc_v[...] = acc
    pltpu.sync_copy(acc_v, out_hbm.at[wid])

  return loss_kernel


_loss_kernel = _build_kernel()


@jax.jit
def kernel(predictions, targets):
  flat_t = (
      predictions.T
      .reshape(N_CLASSES // 8, 8, N_ROWS // 128, 128)
      .transpose(0, 2, 1, 3)
      .reshape(-1)
  )
  tgt = targets.astype(jnp.int32)
  partials = _loss_kernel(flat_t, tgt)
  return -(partials.sum() / jnp.float32(N_ROWS))


# speedup vs baseline: 1.0016x; 1.0016x over previous
import functools

import jax
import jax.numpy as jnp
from jax import lax
from jax.experimental import pallas as pl
from jax.experimental.pallas import tpu as pltpu
from jax.experimental.pallas import tpu_sc as plsc

N_ROWS = 16384
N_CLASSES = 1000
NC = 2
NS = 16
NW = NC * NS
B_PER_W = N_ROWS // NW
CHUNK = 64
N_CHUNKS = B_PER_W // CHUNK
LANES = 16
SUB = CHUNK // LANES


def _build_kernel():
  mesh = plsc.VectorSubcoreMesh(core_axis_name="c", subcore_axis_name="s")

  @functools.partial(
      pl.kernel,
      mesh=mesh,
      out_type=jax.ShapeDtypeStruct((NW, LANES), jnp.float32),
      scratch_types=[
          pltpu.VMEM((B_PER_W,), jnp.int32),
          pltpu.VMEM((N_CHUNKS, CHUNK), jnp.int32),
          pltpu.VMEM((N_CHUNKS, CHUNK), jnp.float32),
          pltpu.VMEM((LANES,), jnp.float32),
          pltpu.SemaphoreType.DMA,
      ],
  )
  def loss_kernel(flat_hbm, tgt_hbm, out_hbm, tgt_v, idx_v, vals_v, acc_v, sem):
    wid = lax.axis_index("s") * NC + lax.axis_index("c")
    base = wid * B_PER_W
    pltpu.sync_copy(tgt_hbm.at[pl.ds(base, B_PER_W)], tgt_v)
    copies = []
    for j in range(N_CHUNKS):
      for k in range(SUB):
        t = tgt_v[pl.ds(j * CHUNK + k * LANES, LANES)]
        i = base + j * CHUNK + k * LANES + lax.iota(jnp.int32, LANES)
        q = lax.shift_right_logical(i * 67109, 26)
        rem = i - q * N_CLASSES
        rr = t + q
        idx_v[j, pl.ds(k * LANES, LANES)] = (
            lax.shift_left(lax.shift_right_logical(rem, 3), 17)
            + lax.shift_left(lax.shift_right_logical(rr, 7), 10)
            + lax.shift_left(lax.bitwise_and(rem, 7), 7)
            + lax.bitwise_and(rr, 127)
        )
      copies.append(
          pltpu.async_copy(flat_hbm.at[idx_v.at[j]], vals_v.at[j], sem))
    acc = jnp.zeros((LANES,), jnp.float32)
    for j in range(N_CHUNKS):
      copies[j].wait()
      for k in range(SUB):
        acc = acc + vals_v[j, pl.ds(k * LANES, LANES)]
    acc_v[...] = acc
    pltpu.sync_copy(acc_v, out_hbm.at[wid])

  return loss_kernel


_loss_kernel = _build_kernel()


@jax.jit
def kernel(predictions, targets):
  flat_t = (
      predictions.T
      .reshape(N_CLASSES // 8, 8, N_ROWS // 128, 128)
      .transpose(0, 2, 1, 3)
      .reshape(-1)
  )
  tgt = targets.astype(jnp.int32)
  partials = _loss_kernel(flat_t, tgt)
  return -(partials.sum() / jnp.float32(N_ROWS))
